# baseline (speedup 1.0000x reference)
import functools

import jax
import jax.numpy as jnp
from jax import lax
from jax.experimental import pallas as pl
from jax.experimental.pallas import tpu as pltpu
from jax.experimental.pallas import tpu_sc as plsc

B = 16384
D = 64
LANES = 16


def _make_sc_kernel(n_workers: int, b_per_w: int):
    mesh = plsc.VectorSubcoreMesh(core_axis_name="c", subcore_axis_name="s")

    @functools.partial(
        pl.kernel,
        mesh=mesh,
        out_type=jax.ShapeDtypeStruct((B, D), jnp.float32),
        compiler_params=pltpu.CompilerParams(use_tc_tiling_on_sc=False),
        scratch_types=[
            pltpu.VMEM((b_per_w,), jnp.int32),
            pltpu.VMEM((b_per_w,), jnp.int32),
            pltpu.VMEM((b_per_w,), jnp.int32),
            pltpu.VMEM((b_per_w, D), jnp.float32),
            pltpu.VMEM((b_per_w, D), jnp.float32),
            pltpu.VMEM((b_per_w, D), jnp.float32),
            pltpu.SemaphoreType.DMA,
            pltpu.SemaphoreType.DMA,
            pltpu.SemaphoreType.DMA,
        ],
    )
    def sc_kernel(pred_hbm, rel_hbm, h_hbm, t_hbm, r_hbm, out_hbm,
                  hidx_v, tidx_v, ridx_v, h_v, t_v, r_v,
                  sem_h, sem_t, sem_r):
        wid = lax.axis_index("s") * 2 + lax.axis_index("c")
        base = wid * b_per_w

        pltpu.sync_copy(h_hbm.at[pl.ds(base, b_per_w)], hidx_v)
        pltpu.sync_copy(t_hbm.at[pl.ds(base, b_per_w)], tidx_v)
        pltpu.sync_copy(r_hbm.at[pl.ds(base, b_per_w)], ridx_v)

        ch = pltpu.async_copy(pred_hbm.at[hidx_v], h_v, sem_h)
        ct = pltpu.async_copy(pred_hbm.at[tidx_v], t_v, sem_t)
        cr = pltpu.async_copy(rel_hbm.at[ridx_v], r_v, sem_r)
        ch.wait()
        ct.wait()
        cr.wait()

        def row(i, carry):
            for j in range(D // LANES):
                sl = pl.ds(j * LANES, LANES)
                h_v[i, sl] = jnp.abs(h_v[i, sl] + r_v[i, sl] - t_v[i, sl])
            return carry

        lax.fori_loop(0, b_per_w, row, 0)

        pltpu.sync_copy(h_v, out_hbm.at[pl.ds(base, b_per_w)])

    return sc_kernel


def kernel(pred_table, rel_table, h_idx, t_idx, r_idx):
    n_workers = 32
    b_per_w = B // n_workers
    sc = _make_sc_kernel(n_workers, b_per_w)
    return sc(
        pred_table,
        rel_table,
        h_idx.astype(jnp.int32),
        t_idx.astype(jnp.int32),
        r_idx.astype(jnp.int32),
    )
